# baseline (speedup 1.0000x reference)
import functools

import jax
import jax.numpy as jnp
from jax import lax
from jax.experimental import pallas as pl
from jax.experimental.pallas import tpu as pltpu
from jax.experimental.pallas import tpu_sc as plsc

N = 10000
E = 320000
H = 128
L = 4

NC = 2
NS = 16
CHUNK = 80
EPT = E // (NC * NS)
NCHUNK = EPT // CHUNK
NP = 10112
ROWS_PT = NP // NS
EW_W = H // 2

_f32 = jnp.float32


def _init_body(x_ref, wn_ref, bn_ref, wm_ref, h_ref, hw_ref):
    h = jnp.dot(x_ref[...], wn_ref[...], preferred_element_type=_f32)
    h = h + bn_ref[...]
    h_ref[...] = h
    hw_ref[...] = jnp.dot(h, wm_ref[...], preferred_element_type=_f32)


def _node_init(x, w_node, b_node, wm0):
    bn = 10
    blk = N // bn
    return pl.pallas_call(
        _init_body,
        grid=(bn,),
        in_specs=[
            pl.BlockSpec((blk, H), lambda i: (i, 0)),
            pl.BlockSpec((H, H), lambda i: (0, 0)),
            pl.BlockSpec((1, H), lambda i: (0, 0)),
            pl.BlockSpec((H, H), lambda i: (0, 0)),
        ],
        out_specs=[
            pl.BlockSpec((blk, H), lambda i: (i, 0)),
            pl.BlockSpec((blk, H), lambda i: (i, 0)),
        ],
        out_shape=[
            jax.ShapeDtypeStruct((N, H), _f32),
            jax.ShapeDtypeStruct((N, H), _f32),
        ],
    )(x, w_node, b_node.reshape(1, H), wm0)


def _ew_body(ea_ref, w2_ref, b2_ref, o0, o1, o2, o3):
    ea = ea_ref[...]
    outs = (o0, o1, o2, o3)
    for l in range(L):
        outs[l][...] = (
            jnp.dot(ea, w2_ref[l], preferred_element_type=_f32) + b2_ref[l]
        ).astype(jnp.bfloat16)


def _edge_terms(edge_attr, w2, b2):
    bn = 160
    blk = E // bn
    d_edge = edge_attr.shape[1]
    return pl.pallas_call(
        _ew_body,
        grid=(bn,),
        in_specs=[
            pl.BlockSpec((blk, d_edge), lambda i: (i, 0)),
            pl.BlockSpec((L, d_edge, H), lambda i: (0, 0, 0)),
            pl.BlockSpec((L, 1, H), lambda i: (0, 0, 0)),
        ],
        out_specs=[pl.BlockSpec((blk, H), lambda i: (i, 0))] * L,
        out_shape=[jax.ShapeDtypeStruct((E, H), jnp.bfloat16)] * L,
    )(edge_attr, w2, b2.reshape(L, 1, H))


def _upd_body(p_ref, h_ref, wu_ref, bu_ref, g_ref, b_ref, wm_ref,
              hn_ref, hw_ref):
    agg = p_ref[0] + p_ref[1]
    u = jnp.dot(agg, wu_ref[...], preferred_element_type=_f32) + bu_ref[...]
    u = jnp.maximum(u, 0.0)
    t = u + h_ref[...]
    mu = jnp.mean(t, axis=-1, keepdims=True)
    var = jnp.mean((t - mu) ** 2, axis=-1, keepdims=True)
    hn = (t - mu) * lax.rsqrt(var + 1e-5) * g_ref[...] + b_ref[...]
    hn_ref[...] = hn
    hw_ref[...] = jnp.dot(hn, wm_ref[...], preferred_element_type=_f32)


def _update(partial, h, wu, bu, gamma, beta, wm_next):
    bn = 10
    blk = N // bn
    return pl.pallas_call(
        _upd_body,
        grid=(bn,),
        in_specs=[
            pl.BlockSpec((NC, blk, H), lambda i: (0, i, 0)),
            pl.BlockSpec((blk, H), lambda i: (i, 0)),
            pl.BlockSpec((H, H), lambda i: (0, 0)),
            pl.BlockSpec((1, H), lambda i: (0, 0)),
            pl.BlockSpec((1, H), lambda i: (0, 0)),
            pl.BlockSpec((1, H), lambda i: (0, 0)),
            pl.BlockSpec((H, H), lambda i: (0, 0)),
        ],
        out_specs=[
            pl.BlockSpec((blk, H), lambda i: (i, 0)),
            pl.BlockSpec((blk, H), lambda i: (i, 0)),
        ],
        out_shape=[
            jax.ShapeDtypeStruct((N, H), _f32),
            jax.ShapeDtypeStruct((N, H), _f32),
        ],
    )(partial, h, wu, bu.reshape(1, H), gamma.reshape(1, H),
      beta.reshape(1, H), wm_next)


NBUF = 3
NROUND = 41
TAIL = NCHUNK - NROUND * NBUF


def _sc_layer_body(src1d, dst1d, hw, ew, zeros_n, out,
                   sidx, didx, ebufs, fbufs, acc, in_sems, g_sems, s_sems):
    c = lax.axis_index("c")
    s = lax.axis_index("s")
    tid = c * NS + s

    rows = pl.ds(s * ROWS_PT, ROWS_PT)
    pltpu.sync_copy(zeros_n.at[rows], acc.at[rows])
    plsc.subcore_barrier()

    eb0 = tid * EPT
    eb0h = tid * (EPT // 2)

    def issue_in(t, b):
        sl = pl.ds(eb0 + t * CHUNK, CHUNK)
        pltpu.async_copy(src1d.at[sl], sidx.at[b], in_sems.at[b])
        pltpu.async_copy(dst1d.at[sl], didx.at[t % (NBUF + 1)],
                         in_sems.at[b])
        slh = pl.ds(eb0h + t * (CHUNK // 2), CHUNK // 2)
        pltpu.async_copy(ew.at[slh], ebufs.at[b], in_sems.at[b])

    def drain_in(b):
        sl0 = pl.ds(0, CHUNK)
        pltpu.make_async_copy(src1d.at[sl0], sidx.at[0], in_sems.at[b]).wait()
        pltpu.make_async_copy(dst1d.at[sl0], didx.at[0], in_sems.at[b]).wait()
        pltpu.make_async_copy(ew.at[pl.ds(0, CHUNK // 2)], ebufs.at[0],
                              in_sems.at[b]).wait()

    def wait_fbuf_bytes(sem_ref):
        pltpu.make_async_copy(hw.at[pl.ds(0, CHUNK)], fbufs.at[0],
                              sem_ref).wait()

    def issue_gather(b):
        pltpu.async_copy(hw.at[sidx.at[b]], fbufs.at[b], g_sems.at[b])

    _HIMASK = jnp.int32(-65536)

    def combine_buf(b):
        def rows_body(r2, carry):
            for k in range(2):
                r = r2 * 2 + k
                for g in range(4):
                    x = ebufs[b, r2, pl.ds(k * 64 + g * 16, 16)]
                    lo = lax.bitcast_convert_type(x << 16, _f32)
                    hi = lax.bitcast_convert_type(x & _HIMASK, _f32)
                    sl0 = pl.ds(g * 32, 16)
                    sl1 = pl.ds(g * 32 + 16, 16)
                    fbufs[b, r, sl0] = jnp.maximum(fbufs[b, r, sl0] + lo, 0.0)
                    fbufs[b, r, sl1] = jnp.maximum(fbufs[b, r, sl1] + hi, 0.0)
            return carry
        lax.fori_loop(0, CHUNK // 2, rows_body, 0)

    def issue_scatter(t, b):
        pltpu.async_copy(fbufs.at[b], acc.at[didx.at[t % (NBUF + 1)]],
                         s_sems.at[b], add=True)

    issue_in(0, 0)
    issue_in(1, 1)
    drain_in(0)
    issue_gather(0)

    def round_body(gi, carry):
        t0 = gi * NBUF
        for b in range(NBUF):
            t = t0 + b
            bn = (b + 1) % NBUF
            bf = (b + 2) % NBUF

            @pl.when(t >= 2)
            def _():
                wait_fbuf_bytes(s_sems.at[bn])

            @pl.when(t <= NCHUNK - 3)
            def _():
                issue_in(t + 2, bf)

            drain_in(bn)
            issue_gather(bn)

            wait_fbuf_bytes(g_sems.at[b])
            combine_buf(b)
            issue_scatter(t, b)
        return carry

    lax.fori_loop(0, NROUND, round_body, 0)

    for t in range(NROUND * NBUF, NCHUNK):
        if t + 1 < NCHUNK:
            wait_fbuf_bytes(s_sems.at[(t + 1) % NBUF])
            drain_in((t + 1) % NBUF)
            issue_gather((t + 1) % NBUF)
        wait_fbuf_bytes(g_sems.at[t % NBUF])
        combine_buf(t % NBUF)
        issue_scatter(t, t % NBUF)
    wait_fbuf_bytes(s_sems.at[(NCHUNK - 3) % NBUF])
    wait_fbuf_bytes(s_sems.at[(NCHUNK - 2) % NBUF])
    wait_fbuf_bytes(s_sems.at[(NCHUNK - 1) % NBUF])
    plsc.subcore_barrier()

    rows2 = pl.ds(s * ROWS_PT, ROWS_PT)
    pltpu.sync_copy(acc.at[rows2], out.at[c, rows2])


_sc_layer = functools.partial(
    pl.kernel,
    out_type=jax.ShapeDtypeStruct((NC, NP, H), _f32),
    mesh=plsc.VectorSubcoreMesh(core_axis_name="c", subcore_axis_name="s"),
    scratch_types=[
        pltpu.VMEM((NBUF, CHUNK), jnp.int32),
        pltpu.VMEM((NBUF + 1, CHUNK), jnp.int32),
        pltpu.VMEM((NBUF, CHUNK // 2, H), jnp.int32),
        pltpu.VMEM((NBUF, CHUNK, H), _f32),
        pltpu.VMEM_SHARED((NP, H), _f32),
        pltpu.SemaphoreType.DMA((NBUF,)),
        pltpu.SemaphoreType.DMA((NBUF,)),
        pltpu.SemaphoreType.DMA((NBUF,)),
    ],
)(_sc_layer_body)


def kernel(x, edge_attr, edge_index, W_node, b_node, W_edge, b_edge,
           W_msg, b_msg, W_upd, b_upd, ln_gamma, ln_beta):
    src1d = edge_index[0]
    dst1d = edge_index[1]

    w2 = jnp.einsum("eh,lhk->lek", W_edge, W_msg,
                    preferred_element_type=_f32)
    b2 = jnp.einsum("h,lhk->lk", b_edge, W_msg,
                    preferred_element_type=_f32) + b_msg
    perm = []
    for g in range(4):
        for r in range(16):
            perm.extend((32 * g + r, 32 * g + 16 + r))
    perm = jnp.array(perm, dtype=jnp.int32)
    w2 = w2[:, :, perm]
    b2 = b2[:, perm]

    h, hw = _node_init(x, W_node, b_node, W_msg[0])
    ew = _edge_terms(edge_attr, w2, b2)
    ew = [jax.lax.bitcast_convert_type(e.reshape(E, EW_W, 2), jnp.int32)
          .reshape(E // 2, H) for e in ew]
    zeros_n = jnp.zeros((NP, H), _f32)

    for i in range(L):
        partial = _sc_layer(src1d, dst1d, hw, ew[i], zeros_n)
        h, hw = _update(partial, h, W_upd[i], b_upd[i],
                        ln_gamma[i], ln_beta[i], W_msg[(i + 1) % L])
    return h


# speedup vs baseline: 4.0186x; 4.0186x over previous
import functools

import jax
import jax.numpy as jnp
from jax import lax
from jax.experimental import pallas as pl
from jax.experimental.pallas import tpu as pltpu
from jax.experimental.pallas import tpu_sc as plsc

N = 10000
E = 320000
H = 128
L = 4

NC = 2
NS = 16
CHUNK = 80
EPT = E // (NC * NS)
NCHUNK = EPT // CHUNK
NP = 10240
ROWS_PT = NP // NS
RCHUNK = 128
NRCHUNK = ROWS_PT // RCHUNK

_f32 = jnp.float32


def _init_body(x_ref, wn_ref, bn_ref, wm_ref, h_ref, hw_ref):
    h = jnp.dot(x_ref[...], wn_ref[...], preferred_element_type=_f32)
    h = h + bn_ref[...]
    h_ref[...] = h
    hw_ref[...] = jnp.dot(h, wm_ref[...], preferred_element_type=_f32)


def _node_init(x, w_node, b_node, wm0):
    bn = 10
    blk = N // bn
    return pl.pallas_call(
        _init_body,
        grid=(bn,),
        in_specs=[
            pl.BlockSpec((blk, H), lambda i: (i, 0)),
            pl.BlockSpec((H, H), lambda i: (0, 0)),
            pl.BlockSpec((1, H), lambda i: (0, 0)),
            pl.BlockSpec((H, H), lambda i: (0, 0)),
        ],
        out_specs=[
            pl.BlockSpec((blk, H), lambda i: (i, 0)),
            pl.BlockSpec((blk, H), lambda i: (i, 0)),
        ],
        out_shape=[
            jax.ShapeDtypeStruct((N, H), _f32),
            jax.ShapeDtypeStruct((N, H), _f32),
        ],
    )(x, w_node, b_node.reshape(1, H), wm0)


def _ew_body(ea_ref, w2_ref, b2_ref, o0, o1, o2, o3):
    ea = ea_ref[...]
    outs = (o0, o1, o2, o3)
    for l in range(L):
        outs[l][...] = (
            jnp.dot(ea, w2_ref[l], preferred_element_type=_f32) + b2_ref[l]
        )


def _edge_terms(edge_attr, w2, b2):
    bn = 160
    blk = E // bn
    d_edge = edge_attr.shape[1]
    return pl.pallas_call(
        _ew_body,
        grid=(bn,),
        in_specs=[
            pl.BlockSpec((blk, d_edge), lambda i: (i, 0)),
            pl.BlockSpec((L, d_edge, H), lambda i: (0, 0, 0)),
            pl.BlockSpec((L, 1, H), lambda i: (0, 0, 0)),
        ],
        out_specs=[pl.BlockSpec((blk, H), lambda i: (i, 0))] * L,
        out_shape=[jax.ShapeDtypeStruct((E, H), _f32)] * L,
    )(edge_attr, w2, b2.reshape(L, 1, H))


def _upd_body(p_ref, h_ref, wu_ref, bu_ref, g_ref, b_ref, wm_ref,
              hn_ref, hw_ref):
    agg = p_ref[0] + p_ref[1]
    u = jnp.dot(agg, wu_ref[...], preferred_element_type=_f32) + bu_ref[...]
    u = jnp.maximum(u, 0.0)
    t = u + h_ref[...]
    mu = jnp.mean(t, axis=-1, keepdims=True)
    var = jnp.mean((t - mu) ** 2, axis=-1, keepdims=True)
    hn = (t - mu) * lax.rsqrt(var + 1e-5) * g_ref[...] + b_ref[...]
    hn_ref[...] = hn
    hw_ref[...] = jnp.dot(hn, wm_ref[...], preferred_element_type=_f32)


def _update(partial, h, wu, bu, gamma, beta, wm_next):
    bn = 10
    blk = N // bn
    return pl.pallas_call(
        _upd_body,
        grid=(bn,),
        in_specs=[
            pl.BlockSpec((NC, blk, H), lambda i: (0, i, 0)),
            pl.BlockSpec((blk, H), lambda i: (i, 0)),
            pl.BlockSpec((H, H), lambda i: (0, 0)),
            pl.BlockSpec((1, H), lambda i: (0, 0)),
            pl.BlockSpec((1, H), lambda i: (0, 0)),
            pl.BlockSpec((1, H), lambda i: (0, 0)),
            pl.BlockSpec((H, H), lambda i: (0, 0)),
        ],
        out_specs=[
            pl.BlockSpec((blk, H), lambda i: (i, 0)),
            pl.BlockSpec((blk, H), lambda i: (i, 0)),
        ],
        out_shape=[
            jax.ShapeDtypeStruct((N, H), _f32),
            jax.ShapeDtypeStruct((N, H), _f32),
        ],
    )(partial, h, wu, bu.reshape(1, H), gamma.reshape(1, H),
      beta.reshape(1, H), wm_next)


NBUF = 4
NROUND = 31
TAIL = NCHUNK - NROUND * NBUF


def _sc_layer_body(src1d, dst1d, hw, ew, zeros_n, out,
                   sidx, didx, bufs, acc, in_sems, g_sems, s_sems):
    c = lax.axis_index("c")
    s = lax.axis_index("s")
    tid = c * NS + s

    for k in range(NRCHUNK):
        rows = pl.ds(s * ROWS_PT + k * RCHUNK, RCHUNK)
        pltpu.sync_copy(zeros_n.at[rows], acc.at[rows])
    plsc.subcore_barrier()

    eb0 = tid * EPT

    def issue_in(t, b):
        sl = pl.ds(eb0 + t * CHUNK, CHUNK)
        pltpu.async_copy(src1d.at[sl], sidx.at[b], in_sems.at[b])
        pltpu.async_copy(dst1d.at[sl], didx.at[b], in_sems.at[b])
        pltpu.async_copy(ew.at[sl], bufs.at[b], in_sems.at[b])

    def drain_in(b):
        sl0 = pl.ds(0, CHUNK)
        pltpu.make_async_copy(src1d.at[sl0], sidx.at[0], in_sems.at[b]).wait()
        pltpu.make_async_copy(dst1d.at[sl0], didx.at[0], in_sems.at[b]).wait()
        pltpu.make_async_copy(ew.at[sl0], bufs.at[0], in_sems.at[b]).wait()

    def wait_buf_bytes(sem_ref):
        pltpu.make_async_copy(ew.at[pl.ds(0, CHUNK)], bufs.at[0],
                              sem_ref).wait()

    def issue_gather(b):
        pltpu.async_copy(hw.at[sidx.at[b]], bufs.at[b], g_sems.at[b],
                         add=True)

    def relu_buf(b):
        def relu_rows(r, carry):
            for k in range(10):
                for j in range(H // 16):
                    sl = pl.ds(j * 16, 16)
                    bufs[b, r * 10 + k, sl] = jnp.maximum(
                        bufs[b, r * 10 + k, sl], 0.0)
            return carry
        lax.fori_loop(0, CHUNK // 10, relu_rows, 0)

    def issue_scatter(b):
        pltpu.async_copy(bufs.at[b], acc.at[didx.at[b]], s_sems.at[b],
                         add=True)

    issue_in(0, 0)
    issue_in(1, 1)
    drain_in(0)
    issue_gather(0)

    def round_body(g, carry):
        t0 = g * NBUF
        for b in range(NBUF):
            t = t0 + b
            bn = (b + 1) % NBUF
            bf = (b + 2) % NBUF

            @pl.when(t >= 2)
            def _():
                wait_buf_bytes(s_sems.at[bf])

            @pl.when(t <= NCHUNK - 3)
            def _():
                issue_in(t + 2, bf)

            drain_in(bn)
            issue_gather(bn)

            wait_buf_bytes(g_sems.at[b])
            relu_buf(b)
            issue_scatter(b)
        return carry

    lax.fori_loop(0, NROUND, round_body, 0)

    for t in range(NROUND * NBUF, NCHUNK):
        wait_buf_bytes(s_sems.at[(t + 2) % NBUF])
        if t + 1 < NCHUNK:
            drain_in((t + 1) % NBUF)
            issue_gather((t + 1) % NBUF)
        wait_buf_bytes(g_sems.at[t % NBUF])
        relu_buf(t % NBUF)
        issue_scatter(t % NBUF)
    wait_buf_bytes(s_sems.at[(NCHUNK - 2) % NBUF])
    wait_buf_bytes(s_sems.at[(NCHUNK - 1) % NBUF])
    plsc.subcore_barrier()

    for k in range(NRCHUNK):
        rows = pl.ds(s * ROWS_PT + k * RCHUNK, RCHUNK)
        pltpu.sync_copy(acc.at[rows], out.at[c, rows])


_sc_layer = functools.partial(
    pl.kernel,
    out_type=jax.ShapeDtypeStruct((NC, NP, H), _f32),
    mesh=plsc.VectorSubcoreMesh(core_axis_name="c", subcore_axis_name="s"),
    scratch_types=[
        pltpu.VMEM((NBUF, CHUNK), jnp.int32),
        pltpu.VMEM((NBUF, CHUNK), jnp.int32),
        pltpu.VMEM((NBUF, CHUNK, H), _f32),
        pltpu.VMEM_SHARED((NP, H), _f32),
        pltpu.SemaphoreType.DMA((NBUF,)),
        pltpu.SemaphoreType.DMA((NBUF,)),
        pltpu.SemaphoreType.DMA((NBUF,)),
    ],
)(_sc_layer_body)


def kernel(x, edge_attr, edge_index, W_node, b_node, W_edge, b_edge,
           W_msg, b_msg, W_upd, b_upd, ln_gamma, ln_beta):
    src1d = edge_index[0]
    dst1d = edge_index[1]

    w2 = jnp.einsum("eh,lhk->lek", W_edge, W_msg,
                    preferred_element_type=_f32)
    b2 = jnp.einsum("h,lhk->lk", b_edge, W_msg,
                    preferred_element_type=_f32) + b_msg

    h, hw = _node_init(x, W_node, b_node, W_msg[0])
    ew = _edge_terms(edge_attr, w2, b2)
    zeros_n = jnp.zeros((NP, H), _f32)

    for i in range(L):
        partial = _sc_layer(src1d, dst1d, hw, ew[i], zeros_n)
        h, hw = _update(partial, h, W_upd[i], b_upd[i],
                        ln_gamma[i], ln_beta[i], W_msg[(i + 1) % L])
    return h


# speedup vs baseline: 4.0540x; 1.0088x over previous
import functools

import jax
import jax.numpy as jnp
from jax import lax
from jax.experimental import pallas as pl
from jax.experimental.pallas import tpu as pltpu
from jax.experimental.pallas import tpu_sc as plsc

N = 10000
E = 320000
H = 128
L = 4

NC = 2
NS = 16
CHUNK = 80
EPT = E // (NC * NS)
NCHUNK = EPT // CHUNK
NP = 10240
ROWS_PT = NP // NS
RCHUNK = 128
NRCHUNK = ROWS_PT // RCHUNK

_f32 = jnp.float32


def _init_body(x_ref, wn_ref, bn_ref, wm_ref, h_ref, hw_ref):
    h = jnp.dot(x_ref[...], wn_ref[...], preferred_element_type=_f32)
    h = h + bn_ref[...]
    h_ref[...] = h
    hw_ref[...] = jnp.dot(h, wm_ref[...], preferred_element_type=_f32)


def _node_init(x, w_node, b_node, wm0):
    bn = 10
    blk = N // bn
    return pl.pallas_call(
        _init_body,
        grid=(bn,),
        in_specs=[
            pl.BlockSpec((blk, H), lambda i: (i, 0)),
            pl.BlockSpec((H, H), lambda i: (0, 0)),
            pl.BlockSpec((1, H), lambda i: (0, 0)),
            pl.BlockSpec((H, H), lambda i: (0, 0)),
        ],
        out_specs=[
            pl.BlockSpec((blk, H), lambda i: (i, 0)),
            pl.BlockSpec((blk, H), lambda i: (i, 0)),
        ],
        out_shape=[
            jax.ShapeDtypeStruct((N, H), _f32),
            jax.ShapeDtypeStruct((N, H), _f32),
        ],
    )(x, w_node, b_node.reshape(1, H), wm0)


def _ew_body(ea_ref, w2_ref, b2_ref, o0, o1, o2, o3):
    ea = ea_ref[...]
    outs = (o0, o1, o2, o3)
    for l in range(L):
        outs[l][...] = (
            jnp.dot(ea, w2_ref[l], preferred_element_type=_f32) + b2_ref[l]
        )


def _edge_terms(edge_attr, w2, b2):
    bn = 160
    blk = E // bn
    d_edge = edge_attr.shape[1]
    return pl.pallas_call(
        _ew_body,
        grid=(bn,),
        in_specs=[
            pl.BlockSpec((blk, d_edge), lambda i: (i, 0)),
            pl.BlockSpec((L, d_edge, H), lambda i: (0, 0, 0)),
            pl.BlockSpec((L, 1, H), lambda i: (0, 0, 0)),
        ],
        out_specs=[pl.BlockSpec((blk, H), lambda i: (i, 0))] * L,
        out_shape=[jax.ShapeDtypeStruct((E, H), _f32)] * L,
    )(edge_attr, w2, b2.reshape(L, 1, H))


def _upd_body(p_ref, h_ref, wu_ref, bu_ref, g_ref, b_ref, wm_ref,
              hn_ref, hw_ref):
    agg = p_ref[0] + p_ref[1]
    u = jnp.dot(agg, wu_ref[...], preferred_element_type=_f32) + bu_ref[...]
    u = jnp.maximum(u, 0.0)
    t = u + h_ref[...]
    mu = jnp.mean(t, axis=-1, keepdims=True)
    var = jnp.mean((t - mu) ** 2, axis=-1, keepdims=True)
    hn = (t - mu) * lax.rsqrt(var + 1e-5) * g_ref[...] + b_ref[...]
    hn_ref[...] = hn
    hw_ref[...] = jnp.dot(hn, wm_ref[...], preferred_element_type=_f32)


def _update(partial, h, wu, bu, gamma, beta, wm_next):
    bn = 10
    blk = N // bn
    return pl.pallas_call(
        _upd_body,
        grid=(bn,),
        in_specs=[
            pl.BlockSpec((NC, blk, H), lambda i: (0, i, 0)),
            pl.BlockSpec((blk, H), lambda i: (i, 0)),
            pl.BlockSpec((H, H), lambda i: (0, 0)),
            pl.BlockSpec((1, H), lambda i: (0, 0)),
            pl.BlockSpec((1, H), lambda i: (0, 0)),
            pl.BlockSpec((1, H), lambda i: (0, 0)),
            pl.BlockSpec((H, H), lambda i: (0, 0)),
        ],
        out_specs=[
            pl.BlockSpec((blk, H), lambda i: (i, 0)),
            pl.BlockSpec((blk, H), lambda i: (i, 0)),
        ],
        out_shape=[
            jax.ShapeDtypeStruct((N, H), _f32),
            jax.ShapeDtypeStruct((N, H), _f32),
        ],
    )(partial, h, wu, bu.reshape(1, H), gamma.reshape(1, H),
      beta.reshape(1, H), wm_next)


NBUF = 4
NROUND = 31
TAIL = NCHUNK - NROUND * NBUF


def _sc_layer_body(src1d, dst1d, hw, ew, zeros_n, out,
                   sidx, didx, bufs, acc, in_sems, g_sems, s_sems):
    c = lax.axis_index("c")
    s = lax.axis_index("s")
    tid = c * NS + s

    for k in range(NRCHUNK):
        rows = pl.ds(s * ROWS_PT + k * RCHUNK, RCHUNK)
        pltpu.sync_copy(zeros_n.at[rows], acc.at[rows])
    plsc.subcore_barrier()

    eb0 = tid * EPT

    def issue_in(t, b):
        sl = pl.ds(eb0 + t * CHUNK, CHUNK)
        pltpu.async_copy(src1d.at[sl], sidx.at[b], in_sems.at[b])
        pltpu.async_copy(dst1d.at[sl], didx.at[b], in_sems.at[b])
        pltpu.async_copy(ew.at[sl], bufs.at[b], in_sems.at[b])

    def drain_in(b):
        sl0 = pl.ds(0, CHUNK)
        pltpu.make_async_copy(src1d.at[sl0], sidx.at[0], in_sems.at[b]).wait()
        pltpu.make_async_copy(dst1d.at[sl0], didx.at[0], in_sems.at[b]).wait()
        pltpu.make_async_copy(ew.at[sl0], bufs.at[0], in_sems.at[b]).wait()

    def wait_buf_bytes(sem_ref):
        pltpu.make_async_copy(ew.at[pl.ds(0, CHUNK)], bufs.at[0],
                              sem_ref).wait()

    def issue_gather(b):
        pltpu.async_copy(hw.at[sidx.at[b]], bufs.at[b], g_sems.at[b],
                         add=True)

    def relu_buf(b):
        def relu_rows(r, carry):
            for k in range(5):
                for j in range(H // 16):
                    sl = pl.ds(j * 16, 16)
                    bufs[b, r * 5 + k, sl] = jnp.maximum(
                        bufs[b, r * 5 + k, sl], 0.0)
            return carry
        lax.fori_loop(0, CHUNK // 5, relu_rows, 0)

    def issue_scatter(b):
        pltpu.async_copy(bufs.at[b], acc.at[didx.at[b]], s_sems.at[b],
                         add=True)

    issue_in(0, 0)
    issue_in(1, 1)
    drain_in(0)
    issue_gather(0)

    def round_body(g, carry):
        t0 = g * NBUF
        for b in range(NBUF):
            t = t0 + b
            bn = (b + 1) % NBUF
            bf = (b + 2) % NBUF

            @pl.when(t >= 2)
            def _():
                wait_buf_bytes(s_sems.at[bf])

            @pl.when(t <= NCHUNK - 3)
            def _():
                issue_in(t + 2, bf)

            drain_in(bn)
            issue_gather(bn)

            wait_buf_bytes(g_sems.at[b])
            relu_buf(b)
            issue_scatter(b)
        return carry

    lax.fori_loop(0, NROUND, round_body, 0)

    for t in range(NROUND * NBUF, NCHUNK):
        wait_buf_bytes(s_sems.at[(t + 2) % NBUF])
        if t + 1 < NCHUNK:
            drain_in((t + 1) % NBUF)
            issue_gather((t + 1) % NBUF)
        wait_buf_bytes(g_sems.at[t % NBUF])
        relu_buf(t % NBUF)
        issue_scatter(t % NBUF)
    wait_buf_bytes(s_sems.at[(NCHUNK - 2) % NBUF])
    wait_buf_bytes(s_sems.at[(NCHUNK - 1) % NBUF])
    plsc.subcore_barrier()

    for k in range(NRCHUNK):
        rows = pl.ds(s * ROWS_PT + k * RCHUNK, RCHUNK)
        pltpu.sync_copy(acc.at[rows], out.at[c, rows])


_sc_layer = functools.partial(
    pl.kernel,
    out_type=jax.ShapeDtypeStruct((NC, NP, H), _f32),
    mesh=plsc.VectorSubcoreMesh(core_axis_name="c", subcore_axis_name="s"),
    scratch_types=[
        pltpu.VMEM((NBUF, CHUNK), jnp.int32),
        pltpu.VMEM((NBUF, CHUNK), jnp.int32),
        pltpu.VMEM((NBUF, CHUNK, H), _f32),
        pltpu.VMEM_SHARED((NP, H), _f32),
        pltpu.SemaphoreType.DMA((NBUF,)),
        pltpu.SemaphoreType.DMA((NBUF,)),
        pltpu.SemaphoreType.DMA((NBUF,)),
    ],
)(_sc_layer_body)


def kernel(x, edge_attr, edge_index, W_node, b_node, W_edge, b_edge,
           W_msg, b_msg, W_upd, b_upd, ln_gamma, ln_beta):
    src1d = edge_index[0]
    dst1d = edge_index[1]

    w2 = jnp.einsum("eh,lhk->lek", W_edge, W_msg,
                    preferred_element_type=_f32)
    b2 = jnp.einsum("h,lhk->lk", b_edge, W_msg,
                    preferred_element_type=_f32) + b_msg

    h, hw = _node_init(x, W_node, b_node, W_msg[0])
    ew = _edge_terms(edge_attr, w2, b2)
    zeros_n = jnp.zeros((NP, H), _f32)

    for i in range(L):
        partial = _sc_layer(src1d, dst1d, hw, ew[i], zeros_n)
        h, hw = _update(partial, h, W_upd[i], b_upd[i],
                        ln_gamma[i], ln_beta[i], W_msg[(i + 1) % L])
    return h
